# baseline (speedup 1.0000x reference)
import dataclasses
import functools

import jax
import jax.numpy as jnp
import numpy as np
from jax import lax
from jax.experimental import pallas as pl
from jax.experimental.pallas import tpu as pltpu
from jax.experimental.pallas import tpu_sc as plsc

OUT_H = 7
OUT_W = 7
SCALE = 0.125
GRID = 2
SLOTS = GRID * GRID * 4
LANES = 16

NUM_CORES = 2
NUM_SUBCORES = 16
NUM_WORKERS = NUM_CORES * NUM_SUBCORES
CHUNK_BINS = 8


def _idxw_body(N, H, W, rois_ref, idx_ref, wt_ref):
    rois = rois_ref[...]
    R = rois.shape[0]
    f32 = jnp.float32
    lane = lax.broadcasted_iota(jnp.int32, (R, OUT_H * OUT_W * SLOTS), 1)
    c = lane % 4
    gx = (lane // 4) % 2
    gy = (lane // 8) % 2
    px = (lane // 16) % OUT_W
    py = lane // (16 * OUT_W)

    bidx = rois[:, 0:1].astype(jnp.int32)
    x1 = rois[:, 1:2] * SCALE - 0.5
    y1 = rois[:, 2:3] * SCALE - 0.5
    x2 = rois[:, 3:4] * SCALE - 0.5
    y2 = rois[:, 4:5] * SCALE - 0.5
    bin_w = (x2 - x1) / OUT_W
    bin_h = (y2 - y1) / OUT_H

    fy = py.astype(f32) + (gy.astype(f32) + 0.5) / GRID
    fx = px.astype(f32) + (gx.astype(f32) + 0.5) / GRID
    ys = y1 + fy * bin_h
    xs = x1 + fx * bin_w

    valid = (ys > -1.0) & (ys < H) & (xs > -1.0) & (xs < W)
    yc = jnp.maximum(ys, 0.0)
    xc = jnp.maximum(xs, 0.0)
    y_low = yc.astype(jnp.int32)
    x_low = xc.astype(jnp.int32)
    ycond = y_low >= H - 1
    xcond = x_low >= W - 1
    y_high = jnp.where(ycond, H - 1, y_low + 1)
    y_low = jnp.where(ycond, H - 1, y_low)
    yc = jnp.where(ycond, y_low.astype(f32), yc)
    x_high = jnp.where(xcond, W - 1, x_low + 1)
    x_low = jnp.where(xcond, W - 1, x_low)
    xc = jnp.where(xcond, x_low.astype(f32), xc)

    ly = yc - y_low.astype(f32)
    lx = xc - x_low.astype(f32)
    wy = jnp.where(c < 2, 1.0 - ly, ly)
    wx = jnp.where(c % 2 == 0, 1.0 - lx, lx)
    wt = wy * wx * jnp.where(valid, 1.0 / (GRID * GRID), 0.0)

    ysel = jnp.where(c < 2, y_low, y_high)
    xsel = jnp.where(c % 2 == 0, x_low, x_high)
    idx = bidx * (H * W) + ysel * W + xsel
    idx_ref[...] = jnp.clip(idx, 0, N * H * W - 1)
    wt_ref[...] = wt


def _make_idxw(R, N, H, W):
    body = functools.partial(_idxw_body, N, H, W)
    return pl.pallas_call(
        body,
        out_shape=[
            jax.ShapeDtypeStruct((R, OUT_H * OUT_W * SLOTS), jnp.int32),
            jax.ShapeDtypeStruct((R, OUT_H * OUT_W * SLOTS), jnp.float32),
        ],
    )


def _sc_body(bins_per_worker, feat_hbm, idx_hbm, wt_hbm, out_hbm,
             idx_all, wt_all, rows0, rows1, out0, out1,
             gsem0, gsem1, osem0, osem1):
    cid = lax.axis_index("c")
    sid = lax.axis_index("s")
    wid = sid * NUM_CORES + cid
    base_bin = wid * bins_per_worker
    gpc = CHUNK_BINS * SLOTS
    nchunks = bins_per_worker // CHUNK_BINS
    C = rows0.shape[1] * 2
    nch = C // LANES

    spw = bins_per_worker * SLOTS
    pltpu.sync_copy(idx_hbm.at[pl.ds(wid * spw, spw)], idx_all)
    pltpu.sync_copy(wt_hbm.at[pl.ds(wid * spw, spw)], wt_all.at[pl.ds(LANES, spw)])

    def issue_gather(t, rows, sem):
        pltpu.async_copy(feat_hbm.at[idx_all.at[pl.ds(t * gpc, gpc)]], rows, sem)

    def wait_gather(rows, sem):
        pltpu.make_async_copy(feat_hbm.at[idx_all.at[pl.ds(0, gpc)]], rows, sem).wait()

    def wait_out(out_v, sem):
        pltpu.make_async_copy(out_v, out_hbm.at[pl.ds(0, CHUNK_BINS)], sem).wait()

    def compute(t, rows, out_v):
        wbase = LANES + t * gpc

        @pl.loop(0, CHUNK_BINS)
        def _bin(j):
            accs = [jnp.zeros((LANES,), jnp.float32) for _ in range(nch)]
            for i in range(SLOTS):
                row = j * SLOTS + i
                wb = plsc.load_gather(
                    wt_all, [jnp.full((LANES,), wbase + row, jnp.int32)])
                wb2 = plsc.pack(wb, wb, format=plsc.PackFormat.INTERLEAVED)
                for m in range(nch // 2):
                    vi = rows[row, pl.ds(m * LANES, LANES)]
                    v = plsc.bitcast(vi, jnp.bfloat16)
                    a, b = plsc.unpack(
                        v * wb2, format=plsc.PackFormat.INTERLEAVED)
                    accs[m] = accs[m] + a
                    accs[m + nch // 2] = accs[m + nch // 2] + b
            for k in range(nch):
                out_v[j, pl.ds(k * LANES, LANES)] = accs[k]

    issue_gather(0, rows0, gsem0)
    issue_gather(1, rows1, gsem1)

    @pl.loop(0, nchunks // 2)
    def _pair(u):
        for phase, rows, out_v, gsem, osem in (
                (0, rows0, out0, gsem0, osem0),
                (1, rows1, out1, gsem1, osem1)):
            t = u * 2 + phase
            wait_gather(rows, gsem)

            @pl.when(u > 0)
            def _():
                wait_out(out_v, osem)

            compute(t, rows, out_v)
            pltpu.async_copy(out_v, out_hbm.at[pl.ds(base_bin + t * CHUNK_BINS,
                                                     CHUNK_BINS)], osem)

            @pl.when(u < nchunks // 2 - 1)
            def _():
                issue_gather(t + 2, rows, gsem)

    wait_out(out0, osem0)
    wait_out(out1, osem1)


def _make_sc(total_bins, bins_per_worker, C):
    mesh = plsc.VectorSubcoreMesh(core_axis_name="c", subcore_axis_name="s")
    gpc = CHUNK_BINS * SLOTS
    spw = bins_per_worker * SLOTS
    cp = pltpu.CompilerParams()
    if "needs_layout_passes" in pltpu.CompilerParams.__dataclass_fields__:
        cp = dataclasses.replace(cp, needs_layout_passes=False)
    return pl.kernel(
        functools.partial(_sc_body, bins_per_worker),
        out_type=jax.ShapeDtypeStruct((total_bins, C), jnp.float32),
        mesh=mesh,
        scratch_types=[
            pltpu.VMEM((spw,), jnp.int32),
            pltpu.VMEM((spw + LANES,), jnp.float32),
            pltpu.VMEM((gpc, C // 2), jnp.int32),
            pltpu.VMEM((gpc, C // 2), jnp.int32),
            pltpu.VMEM((CHUNK_BINS, C), jnp.float32),
            pltpu.VMEM((CHUNK_BINS, C), jnp.float32),
            pltpu.SemaphoreType.DMA,
            pltpu.SemaphoreType.DMA,
            pltpu.SemaphoreType.DMA,
            pltpu.SemaphoreType.DMA,
        ],
        compiler_params=cp,
    )


def _pack_body(lo_ref, hi_ref, out_ref):
    def to_bits(v):
        b = lax.bitcast_convert_type(v.astype(jnp.bfloat16), jnp.int16)
        return b.astype(jnp.int32) & 0xFFFF

    out_ref[...] = to_bits(lo_ref[...]) | (to_bits(hi_ref[...]) << 16)


def _make_pack(N, C, H, W):
    hc = C // 2
    return pl.pallas_call(
        _pack_body,
        grid=(N,),
        in_specs=[
            pl.BlockSpec((1, hc, H, W), lambda n: (n, 0, 0, 0)),
            pl.BlockSpec((1, hc, H, W), lambda n: (n, 1, 0, 0)),
        ],
        out_specs=pl.BlockSpec((1, hc, H, W), lambda n: (n, 0, 0, 0)),
        out_shape=jax.ShapeDtypeStruct((N, hc, H, W), jnp.int32),
    )


def _pack_table(input, N, C, H, W):
    packed = _make_pack(N, C, H, W)(input, input)
    return jnp.transpose(packed, (0, 2, 3, 1)).reshape(N * H * W, C // 2)


ROIS_PER_OTR = 8


def _otr_body(x_ref, out_ref):
    nb = OUT_H * OUT_W
    xt = jnp.transpose(x_ref[...], (1, 0))
    for k in range(ROIS_PER_OTR):
        out_ref[k] = xt[:, k * nb:(k + 1) * nb]


def _make_otr(total_bins, R, C):
    nb = OUT_H * OUT_W
    blk = ROIS_PER_OTR * nb
    return pl.pallas_call(
        _otr_body,
        grid=(R // ROIS_PER_OTR,),
        in_specs=[pl.BlockSpec((blk, C), lambda r: (r, 0))],
        out_specs=pl.BlockSpec((ROIS_PER_OTR, C, nb), lambda r: (r, 0, 0)),
        out_shape=jax.ShapeDtypeStruct((R, C, nb), jnp.float32),
    )


def kernel(input, rois):
    N, C, H, W = input.shape
    R = rois.shape[0]
    bins = R * OUT_H * OUT_W
    step = NUM_WORKERS * CHUNK_BINS
    total_bins = ((bins + step - 1) // step) * step
    bins_per_worker = total_bins // NUM_WORKERS

    feat = _pack_table(input, N, C, H, W)
    idx, wt = _make_idxw(R, N, H, W)(rois)
    pad = total_bins * SLOTS - bins * SLOTS
    idx_f = jnp.pad(idx.reshape(-1), (0, pad))
    wt_f = jnp.pad(wt.reshape(-1), (0, pad))
    out_flat = _make_sc(total_bins, bins_per_worker, C)(feat, idx_f, wt_f)
    out = _make_otr(total_bins, R, C)(out_flat)
    return out.reshape(R, C, OUT_H, OUT_W)


# speedup vs baseline: 1.0300x; 1.0300x over previous
import dataclasses
import functools

import jax
import jax.numpy as jnp
import numpy as np
from jax import lax
from jax.experimental import pallas as pl
from jax.experimental.pallas import tpu as pltpu
from jax.experimental.pallas import tpu_sc as plsc

OUT_H = 7
OUT_W = 7
SCALE = 0.125
GRID = 2
SLOTS = GRID * GRID * 4
LANES = 16

NUM_CORES = 2
NUM_SUBCORES = 16
NUM_WORKERS = NUM_CORES * NUM_SUBCORES
CHUNK_BINS = 8


def _idxw_body(N, H, W, rois_ref, idx_ref, wt_ref):
    rois = rois_ref[...]
    R = rois.shape[0]
    f32 = jnp.float32
    lane = lax.broadcasted_iota(jnp.int32, (R, OUT_H * OUT_W * SLOTS), 1)
    c = lane % 4
    gx = (lane // 4) % 2
    gy = (lane // 8) % 2
    px = (lane // 16) % OUT_W
    py = lane // (16 * OUT_W)

    bidx = rois[:, 0:1].astype(jnp.int32)
    x1 = rois[:, 1:2] * SCALE - 0.5
    y1 = rois[:, 2:3] * SCALE - 0.5
    x2 = rois[:, 3:4] * SCALE - 0.5
    y2 = rois[:, 4:5] * SCALE - 0.5
    bin_w = (x2 - x1) / OUT_W
    bin_h = (y2 - y1) / OUT_H

    fy = py.astype(f32) + (gy.astype(f32) + 0.5) / GRID
    fx = px.astype(f32) + (gx.astype(f32) + 0.5) / GRID
    ys = y1 + fy * bin_h
    xs = x1 + fx * bin_w

    valid = (ys > -1.0) & (ys < H) & (xs > -1.0) & (xs < W)
    yc = jnp.maximum(ys, 0.0)
    xc = jnp.maximum(xs, 0.0)
    y_low = yc.astype(jnp.int32)
    x_low = xc.astype(jnp.int32)
    ycond = y_low >= H - 1
    xcond = x_low >= W - 1
    y_high = jnp.where(ycond, H - 1, y_low + 1)
    y_low = jnp.where(ycond, H - 1, y_low)
    yc = jnp.where(ycond, y_low.astype(f32), yc)
    x_high = jnp.where(xcond, W - 1, x_low + 1)
    x_low = jnp.where(xcond, W - 1, x_low)
    xc = jnp.where(xcond, x_low.astype(f32), xc)

    ly = yc - y_low.astype(f32)
    lx = xc - x_low.astype(f32)
    wy = jnp.where(c < 2, 1.0 - ly, ly)
    wx = jnp.where(c % 2 == 0, 1.0 - lx, lx)
    wt = wy * wx * jnp.where(valid, 1.0 / (GRID * GRID), 0.0)

    ysel = jnp.where(c < 2, y_low, y_high)
    xsel = jnp.where(c % 2 == 0, x_low, x_high)
    idx = bidx * (H * W) + ysel * W + xsel
    idx_ref[...] = jnp.clip(idx, 0, N * H * W - 1)
    wt_ref[...] = wt


def _make_idxw(R, N, H, W):
    body = functools.partial(_idxw_body, N, H, W)
    return pl.pallas_call(
        body,
        out_shape=[
            jax.ShapeDtypeStruct((R, OUT_H * OUT_W * SLOTS), jnp.int32),
            jax.ShapeDtypeStruct((R, OUT_H * OUT_W * SLOTS), jnp.float32),
        ],
    )


def _sc_body(bins_per_worker, feat_hbm, idx_hbm, wt_hbm, out_hbm,
             idx_all, wt_all, rows0, rows1, out0, out1,
             gsem0, gsem1, osem0, osem1):
    cid = lax.axis_index("c")
    sid = lax.axis_index("s")
    wid = sid * NUM_CORES + cid
    base_bin = wid * bins_per_worker
    gpc = CHUNK_BINS * SLOTS
    nchunks = bins_per_worker // CHUNK_BINS
    C = rows0.shape[1] * 2
    nch = C // LANES

    spw = bins_per_worker * SLOTS
    pltpu.sync_copy(idx_hbm.at[pl.ds(wid * spw, spw)], idx_all)
    pltpu.sync_copy(wt_hbm.at[pl.ds(wid * spw, spw)], wt_all.at[pl.ds(LANES, spw)])

    def issue_gather(t, rows, sem):
        pltpu.async_copy(feat_hbm.at[idx_all.at[pl.ds(t * gpc, gpc)]], rows, sem)

    def wait_gather(rows, sem):
        pltpu.make_async_copy(feat_hbm.at[idx_all.at[pl.ds(0, gpc)]], rows, sem).wait()

    def wait_out(out_v, sem):
        pltpu.make_async_copy(out_v, out_hbm.at[pl.ds(0, CHUNK_BINS)], sem).wait()

    def compute(t, rows, out_v):
        wbase = LANES + t * gpc

        @pl.loop(0, CHUNK_BINS)
        def _bin(j):
            accs = [jnp.zeros((LANES,), jnp.float32) for _ in range(nch)]
            for i in range(SLOTS):
                row = j * SLOTS + i
                wb = plsc.load_gather(
                    wt_all, [jnp.full((LANES,), wbase + row, jnp.int32)])
                wb2 = plsc.pack(wb, wb, format=plsc.PackFormat.INTERLEAVED)
                for m in range(nch // 2):
                    vi = rows[row, pl.ds(m * LANES, LANES)]
                    v = plsc.bitcast(vi, jnp.bfloat16)
                    a, b = plsc.unpack(
                        v * wb2, format=plsc.PackFormat.INTERLEAVED)
                    accs[m] = accs[m] + a
                    accs[m + nch // 2] = accs[m + nch // 2] + b
            for k in range(nch):
                out_v[j, pl.ds(k * LANES, LANES)] = accs[k]

    issue_gather(0, rows0, gsem0)
    issue_gather(1, rows1, gsem1)

    @pl.loop(0, nchunks // 2)
    def _pair(u):
        for phase, rows, out_v, gsem, osem in (
                (0, rows0, out0, gsem0, osem0),
                (1, rows1, out1, gsem1, osem1)):
            t = u * 2 + phase
            wait_gather(rows, gsem)

            @pl.when(u > 0)
            def _():
                wait_out(out_v, osem)

            compute(t, rows, out_v)
            pltpu.async_copy(out_v, out_hbm.at[pl.ds(base_bin + t * CHUNK_BINS,
                                                     CHUNK_BINS)], osem)

            @pl.when(u < nchunks // 2 - 1)
            def _():
                issue_gather(t + 2, rows, gsem)

    wait_out(out0, osem0)
    wait_out(out1, osem1)


def _make_sc(total_bins, bins_per_worker, C):
    mesh = plsc.VectorSubcoreMesh(core_axis_name="c", subcore_axis_name="s")
    gpc = CHUNK_BINS * SLOTS
    spw = bins_per_worker * SLOTS
    cp = pltpu.CompilerParams()
    if "needs_layout_passes" in pltpu.CompilerParams.__dataclass_fields__:
        cp = dataclasses.replace(cp, needs_layout_passes=False)
    return pl.kernel(
        functools.partial(_sc_body, bins_per_worker),
        out_type=jax.ShapeDtypeStruct((total_bins, C), jnp.float32),
        mesh=mesh,
        scratch_types=[
            pltpu.VMEM((spw,), jnp.int32),
            pltpu.VMEM((spw + LANES,), jnp.float32),
            pltpu.VMEM((gpc, C // 2), jnp.int32),
            pltpu.VMEM((gpc, C // 2), jnp.int32),
            pltpu.VMEM((CHUNK_BINS, C), jnp.float32),
            pltpu.VMEM((CHUNK_BINS, C), jnp.float32),
            pltpu.SemaphoreType.DMA,
            pltpu.SemaphoreType.DMA,
            pltpu.SemaphoreType.DMA,
            pltpu.SemaphoreType.DMA,
        ],
        compiler_params=cp,
    )


def _pack_body(lo_ref, hi_ref, out_ref):
    def to_bits(v):
        b = lax.bitcast_convert_type(v.astype(jnp.bfloat16), jnp.int16)
        return b.astype(jnp.int32) & 0xFFFF

    packed = to_bits(lo_ref[0]) | (to_bits(hi_ref[0]) << 16)
    out_ref[...] = jnp.transpose(packed, (1, 0))


def _make_pack(N, C, HW):
    hc = C // 2
    return pl.pallas_call(
        _pack_body,
        grid=(N,),
        in_specs=[
            pl.BlockSpec((1, hc, HW), lambda n: (n, 0, 0)),
            pl.BlockSpec((1, hc, HW), lambda n: (n, 1, 0)),
        ],
        out_specs=pl.BlockSpec((HW, hc), lambda n: (n, 0)),
        out_shape=jax.ShapeDtypeStruct((N * HW, hc), jnp.int32),
    )


def _pack_table(input, N, C, H, W):
    x = input.reshape(N, C, H * W)
    return _make_pack(N, C, H * W)(x, x)


ROIS_PER_OTR = 8


def _otr_body(x_ref, out_ref):
    nb = OUT_H * OUT_W
    xt = jnp.transpose(x_ref[...], (1, 0))
    for k in range(ROIS_PER_OTR):
        out_ref[k] = xt[:, k * nb:(k + 1) * nb]


def _make_otr(total_bins, R, C):
    nb = OUT_H * OUT_W
    blk = ROIS_PER_OTR * nb
    return pl.pallas_call(
        _otr_body,
        grid=(R // ROIS_PER_OTR,),
        in_specs=[pl.BlockSpec((blk, C), lambda r: (r, 0))],
        out_specs=pl.BlockSpec((ROIS_PER_OTR, C, nb), lambda r: (r, 0, 0)),
        out_shape=jax.ShapeDtypeStruct((R, C, nb), jnp.float32),
    )


def kernel(input, rois):
    N, C, H, W = input.shape
    R = rois.shape[0]
    bins = R * OUT_H * OUT_W
    step = NUM_WORKERS * CHUNK_BINS
    total_bins = ((bins + step - 1) // step) * step
    bins_per_worker = total_bins // NUM_WORKERS

    feat = _pack_table(input, N, C, H, W)
    idx, wt = _make_idxw(R, N, H, W)(rois)
    pad = total_bins * SLOTS - bins * SLOTS
    idx_f = jnp.pad(idx.reshape(-1), (0, pad))
    wt_f = jnp.pad(wt.reshape(-1), (0, pad))
    out_flat = _make_sc(total_bins, bins_per_worker, C)(feat, idx_f, wt_f)
    out = _make_otr(total_bins, R, C)(out_flat)
    return out.reshape(R, C, OUT_H, OUT_W)


# speedup vs baseline: 1.7450x; 1.6942x over previous
import dataclasses
import functools

import jax
import jax.numpy as jnp
import numpy as np
from jax import lax
from jax.experimental import pallas as pl
from jax.experimental.pallas import tpu as pltpu
from jax.experimental.pallas import tpu_sc as plsc

OUT_H = 7
OUT_W = 7
SCALE = 0.125
GRID = 2
SLOTS = GRID * GRID * 4
LANES = 16

NUM_CORES = 2
NUM_SUBCORES = 16
NUM_WORKERS = NUM_CORES * NUM_SUBCORES
CHUNK_BINS = 8


def _idxw_body(N, H, W, rois_ref, idx_ref, wt_ref):
    rois = rois_ref[...]
    R = rois.shape[0]
    f32 = jnp.float32
    lane = lax.broadcasted_iota(jnp.int32, (R, OUT_H * OUT_W * SLOTS), 1)
    c = lane % 4
    gx = (lane // 4) % 2
    gy = (lane // 8) % 2
    px = (lane // 16) % OUT_W
    py = lane // (16 * OUT_W)

    bidx = rois[:, 0:1].astype(jnp.int32)
    x1 = rois[:, 1:2] * SCALE - 0.5
    y1 = rois[:, 2:3] * SCALE - 0.5
    x2 = rois[:, 3:4] * SCALE - 0.5
    y2 = rois[:, 4:5] * SCALE - 0.5
    bin_w = (x2 - x1) / OUT_W
    bin_h = (y2 - y1) / OUT_H

    fy = py.astype(f32) + (gy.astype(f32) + 0.5) / GRID
    fx = px.astype(f32) + (gx.astype(f32) + 0.5) / GRID
    ys = y1 + fy * bin_h
    xs = x1 + fx * bin_w

    valid = (ys > -1.0) & (ys < H) & (xs > -1.0) & (xs < W)
    yc = jnp.maximum(ys, 0.0)
    xc = jnp.maximum(xs, 0.0)
    y_low = yc.astype(jnp.int32)
    x_low = xc.astype(jnp.int32)
    ycond = y_low >= H - 1
    xcond = x_low >= W - 1
    y_high = jnp.where(ycond, H - 1, y_low + 1)
    y_low = jnp.where(ycond, H - 1, y_low)
    yc = jnp.where(ycond, y_low.astype(f32), yc)
    x_high = jnp.where(xcond, W - 1, x_low + 1)
    x_low = jnp.where(xcond, W - 1, x_low)
    xc = jnp.where(xcond, x_low.astype(f32), xc)

    ly = yc - y_low.astype(f32)
    lx = xc - x_low.astype(f32)
    wy = jnp.where(c < 2, 1.0 - ly, ly)
    wx = jnp.where(c % 2 == 0, 1.0 - lx, lx)
    wt = wy * wx * jnp.where(valid, 1.0 / (GRID * GRID), 0.0)

    ysel = jnp.where(c < 2, y_low, y_high)
    xsel = jnp.where(c % 2 == 0, x_low, x_high)
    idx = bidx * (H * W) + ysel * W + xsel
    idx_ref[...] = jnp.clip(idx, 0, N * H * W - 1)
    wt_ref[...] = wt


def _make_idxw(R, N, H, W):
    body = functools.partial(_idxw_body, N, H, W)
    return pl.pallas_call(
        body,
        out_shape=[
            jax.ShapeDtypeStruct((R, OUT_H * OUT_W * SLOTS), jnp.int32),
            jax.ShapeDtypeStruct((R, OUT_H * OUT_W * SLOTS), jnp.float32),
        ],
    )


def _sc_body(total_chunks, max_chunks, feat_hbm, idx_hbm, wt_hbm, out_hbm,
             idx_all, wt_all, rows0, rows1, out0, out1,
             gsem0, gsem1, osem0, osem1):
    cid = lax.axis_index("c")
    sid = lax.axis_index("s")
    wid = sid * NUM_CORES + cid
    gpc = CHUNK_BINS * SLOTS
    C = rows0.shape[1] * 2
    nch = C // LANES

    q = total_chunks // NUM_WORKERS
    rem = total_chunks - q * NUM_WORKERS
    nchunks = q + jnp.where(wid < rem, 1, 0)
    base_chunk = q * wid + jnp.minimum(wid, rem)

    spw = max_chunks * gpc
    pltpu.sync_copy(idx_hbm.at[pl.ds(base_chunk * gpc, spw)], idx_all)
    pltpu.sync_copy(wt_hbm.at[pl.ds(base_chunk * gpc, spw)],
                    wt_all.at[pl.ds(LANES, spw)])

    def issue_gather(t, rows, sem):
        pltpu.async_copy(feat_hbm.at[idx_all.at[pl.ds(t * gpc, gpc)]], rows, sem)

    def wait_gather(rows, sem):
        pltpu.make_async_copy(feat_hbm.at[idx_all.at[pl.ds(0, gpc)]], rows, sem).wait()

    def wait_out(out_v, sem):
        pltpu.make_async_copy(out_v, out_hbm.at[pl.ds(0, CHUNK_BINS)], sem).wait()

    def compute(t, rows, out_v):
        wbase = LANES + t * gpc

        @pl.loop(0, CHUNK_BINS)
        def _bin(j):
            accs = [jnp.zeros((LANES,), jnp.float32) for _ in range(nch)]
            for i in range(SLOTS):
                row = j * SLOTS + i
                wb = plsc.load_gather(
                    wt_all, [jnp.full((LANES,), wbase + row, jnp.int32)])
                wb2 = plsc.pack(wb, wb, format=plsc.PackFormat.INTERLEAVED)
                for m in range(nch // 2):
                    vi = rows[row, pl.ds(m * LANES, LANES)]
                    v = plsc.bitcast(vi, jnp.bfloat16)
                    a, b = plsc.unpack(
                        v * wb2, format=plsc.PackFormat.INTERLEAVED)
                    accs[m] = accs[m] + a
                    accs[m + nch // 2] = accs[m + nch // 2] + b
            for k in range(nch):
                out_v[j, pl.ds(k * LANES, LANES)] = accs[k]

    def write_out(t, out_v, sem):
        pltpu.async_copy(
            out_v,
            out_hbm.at[pl.ds((base_chunk + t) * CHUNK_BINS, CHUNK_BINS)], sem)

    issue_gather(0, rows0, gsem0)
    issue_gather(1, rows1, gsem1)

    @pl.loop(0, nchunks // 2)
    def _pair(u):
        for phase, rows, out_v, gsem, osem in (
                (0, rows0, out0, gsem0, osem0),
                (1, rows1, out1, gsem1, osem1)):
            t = u * 2 + phase
            wait_gather(rows, gsem)

            @pl.when(u > 0)
            def _():
                wait_out(out_v, osem)

            compute(t, rows, out_v)
            write_out(t, out_v, osem)

            @pl.when(t + 2 < nchunks)
            def _():
                issue_gather(t + 2, rows, gsem)

    @pl.when(nchunks % 2 == 1)
    def _tail():
        t = nchunks - 1
        wait_gather(rows0, gsem0)
        wait_out(out0, osem0)
        compute(t, rows0, out0)
        write_out(t, out0, osem0)

    wait_out(out0, osem0)
    wait_out(out1, osem1)


def _make_sc(total_bins, C):
    mesh = plsc.VectorSubcoreMesh(core_axis_name="c", subcore_axis_name="s")
    gpc = CHUNK_BINS * SLOTS
    total_chunks = total_bins // CHUNK_BINS
    max_chunks = (total_chunks + NUM_WORKERS - 1) // NUM_WORKERS
    spw = max_chunks * gpc
    cp = pltpu.CompilerParams()
    if "needs_layout_passes" in pltpu.CompilerParams.__dataclass_fields__:
        cp = dataclasses.replace(cp, needs_layout_passes=False)
    return pl.kernel(
        functools.partial(_sc_body, total_chunks, max_chunks),
        out_type=jax.ShapeDtypeStruct((total_bins, C), jnp.float32),
        mesh=mesh,
        scratch_types=[
            pltpu.VMEM((spw,), jnp.int32),
            pltpu.VMEM((spw + LANES,), jnp.float32),
            pltpu.VMEM((gpc, C // 2), jnp.int32),
            pltpu.VMEM((gpc, C // 2), jnp.int32),
            pltpu.VMEM((CHUNK_BINS, C), jnp.float32),
            pltpu.VMEM((CHUNK_BINS, C), jnp.float32),
            pltpu.SemaphoreType.DMA,
            pltpu.SemaphoreType.DMA,
            pltpu.SemaphoreType.DMA,
            pltpu.SemaphoreType.DMA,
        ],
        compiler_params=cp,
    )


def _pack_body(lo_ref, hi_ref, out_ref):
    def to_bits(v):
        b = lax.bitcast_convert_type(v.astype(jnp.bfloat16), jnp.int16)
        return b.astype(jnp.int32) & 0xFFFF

    out_ref[...] = to_bits(lo_ref[...]) | (to_bits(hi_ref[...]) << 16)


def _make_pack(NHW, C, BLK):
    hc = C // 2
    return pl.pallas_call(
        _pack_body,
        grid=(NHW // BLK,),
        in_specs=[
            pl.BlockSpec((BLK, hc), lambda i: (i, 0)),
            pl.BlockSpec((BLK, hc), lambda i: (i, 1)),
        ],
        out_specs=pl.BlockSpec((BLK, hc), lambda i: (i, 0)),
        out_shape=jax.ShapeDtypeStruct((NHW, hc), jnp.int32),
    )


def _pack_table(input, N, C, H, W):
    nhwc = jnp.transpose(input, (0, 2, 3, 1)).reshape(N * H * W, C)
    return _make_pack(N * H * W, C, 2000)(nhwc, nhwc)


def kernel(input, rois):
    N, C, H, W = input.shape
    R = rois.shape[0]
    nb = OUT_H * OUT_W
    bins = R * nb
    total_chunks = bins // CHUNK_BINS
    max_chunks = (total_chunks + NUM_WORKERS - 1) // NUM_WORKERS

    feat = _pack_table(input, N, C, H, W)
    idx, wt = _make_idxw(R, N, H, W)(rois)
    pad = max_chunks * NUM_WORKERS * CHUNK_BINS * SLOTS - bins * SLOTS
    idx_bm = jnp.pad(
        idx.reshape(R, nb, SLOTS).transpose(1, 0, 2).reshape(-1), (0, pad))
    wt_bm = jnp.pad(
        wt.reshape(R, nb, SLOTS).transpose(1, 0, 2).reshape(-1), (0, pad))
    out_flat = _make_sc(bins, C)(feat, idx_bm, wt_bm)
    out = out_flat.reshape(OUT_H, OUT_W, R, C)
    return jnp.transpose(out, (2, 3, 0, 1))


# speedup vs baseline: 2.0193x; 1.1571x over previous
import dataclasses
import functools

import jax
import jax.numpy as jnp
import numpy as np
from jax import lax
from jax.experimental import pallas as pl
from jax.experimental.pallas import tpu as pltpu
from jax.experimental.pallas import tpu_sc as plsc

OUT_H = 7
OUT_W = 7
SCALE = 0.125
GRID = 2
SLOTS = GRID * GRID * 4
LANES = 16

NUM_CORES = 2
NUM_SUBCORES = 16
NUM_WORKERS = NUM_CORES * NUM_SUBCORES
CHUNK_BINS = 8


def _idxw_body(N, H, W, cols_ref, idx_ref, wt_ref):
    f32 = jnp.float32
    L = cols_ref.shape[1]
    shape = (OUT_H * OUT_W, L)
    lane = lax.broadcasted_iota(jnp.int32, shape, 1)
    slot = lane % SLOTS
    c = slot % 4
    gx = (slot // 4) % 2
    gy = (slot // 8) % 2
    brow = lax.broadcasted_iota(jnp.int32, shape, 0)
    py = brow // OUT_W
    px = brow % OUT_W

    bidx = cols_ref[0:1, :].astype(jnp.int32)
    x1 = cols_ref[1:2, :] * SCALE - 0.5
    y1 = cols_ref[2:3, :] * SCALE - 0.5
    x2 = cols_ref[3:4, :] * SCALE - 0.5
    y2 = cols_ref[4:5, :] * SCALE - 0.5
    bin_w = (x2 - x1) / OUT_W
    bin_h = (y2 - y1) / OUT_H

    fy = py.astype(f32) + (gy.astype(f32) + 0.5) / GRID
    fx = px.astype(f32) + (gx.astype(f32) + 0.5) / GRID
    ys = y1 + fy * bin_h
    xs = x1 + fx * bin_w

    valid = (ys > -1.0) & (ys < H) & (xs > -1.0) & (xs < W)
    yc = jnp.maximum(ys, 0.0)
    xc = jnp.maximum(xs, 0.0)
    y_low = yc.astype(jnp.int32)
    x_low = xc.astype(jnp.int32)
    ycond = y_low >= H - 1
    xcond = x_low >= W - 1
    y_high = jnp.where(ycond, H - 1, y_low + 1)
    y_low = jnp.where(ycond, H - 1, y_low)
    yc = jnp.where(ycond, y_low.astype(f32), yc)
    x_high = jnp.where(xcond, W - 1, x_low + 1)
    x_low = jnp.where(xcond, W - 1, x_low)
    xc = jnp.where(xcond, x_low.astype(f32), xc)

    ly = yc - y_low.astype(f32)
    lx = xc - x_low.astype(f32)
    wy = jnp.where(c < 2, 1.0 - ly, ly)
    wx = jnp.where(c % 2 == 0, 1.0 - lx, lx)
    wt = wy * wx * jnp.where(valid, 1.0 / (GRID * GRID), 0.0)

    ysel = jnp.where(c < 2, y_low, y_high)
    xsel = jnp.where(c % 2 == 0, x_low, x_high)
    idx = bidx * (H * W) + ysel * W + xsel
    idx_ref[...] = jnp.clip(idx, 0, N * H * W - 1)
    wt_ref[...] = wt


def _make_idxw(R, N, H, W):
    body = functools.partial(_idxw_body, N, H, W)
    return pl.pallas_call(
        body,
        out_shape=[
            jax.ShapeDtypeStruct((OUT_H * OUT_W, R * SLOTS), jnp.int32),
            jax.ShapeDtypeStruct((OUT_H * OUT_W, R * SLOTS), jnp.float32),
        ],
    )


def _sc_body(total_chunks, max_chunks, feat_hbm, idx_hbm, wt_hbm, out_hbm,
             idx_all, wt_all, rows0, rows1, out0, out1,
             gsem0, gsem1, osem0, osem1):
    cid = lax.axis_index("c")
    sid = lax.axis_index("s")
    wid = sid * NUM_CORES + cid
    gpc = CHUNK_BINS * SLOTS
    C = rows0.shape[1] * 2
    nch = C // LANES

    q = total_chunks // NUM_WORKERS
    rem = total_chunks - q * NUM_WORKERS
    nchunks = q + jnp.where(wid < rem, 1, 0)
    base_chunk = q * wid + jnp.minimum(wid, rem)

    spw = max_chunks * gpc
    pltpu.sync_copy(idx_hbm.at[pl.ds(base_chunk * gpc, spw)], idx_all)
    pltpu.sync_copy(wt_hbm.at[pl.ds(base_chunk * gpc, spw)],
                    wt_all.at[pl.ds(LANES, spw)])

    def issue_gather(t, rows, sem):
        pltpu.async_copy(feat_hbm.at[idx_all.at[pl.ds(t * gpc, gpc)]], rows, sem)

    def wait_gather(rows, sem):
        pltpu.make_async_copy(feat_hbm.at[idx_all.at[pl.ds(0, gpc)]], rows, sem).wait()

    def wait_out(out_v, sem):
        pltpu.make_async_copy(out_v, out_hbm.at[pl.ds(0, CHUNK_BINS)], sem).wait()

    def compute(t, rows, out_v):
        wbase = LANES + t * gpc

        @pl.loop(0, CHUNK_BINS)
        def _bin(j):
            accs = [jnp.zeros((LANES,), jnp.float32) for _ in range(nch)]
            for s in range(SLOTS // 4):
                prods = [[None] * 4 for _ in range(nch // 2)]
                for c in range(4):
                    row = j * SLOTS + s * 4 + c
                    wb = plsc.load_gather(
                        wt_all, [jnp.full((LANES,), wbase + row, jnp.int32)])
                    wb2 = plsc.pack(wb, wb, format=plsc.PackFormat.INTERLEAVED)
                    for m in range(nch // 2):
                        vi = rows[row, pl.ds(m * LANES, LANES)]
                        prods[m][c] = plsc.bitcast(vi, jnp.bfloat16) * wb2
                for m in range(nch // 2):
                    pm = prods[m]
                    acc_bf = (pm[0] + pm[1]) + (pm[2] + pm[3])
                    a, b = plsc.unpack(
                        acc_bf, format=plsc.PackFormat.INTERLEAVED)
                    accs[m] = accs[m] + a
                    accs[m + nch // 2] = accs[m + nch // 2] + b
            for k in range(nch):
                out_v[j, pl.ds(k * LANES, LANES)] = accs[k]

    def write_out(t, out_v, sem):
        pltpu.async_copy(
            out_v,
            out_hbm.at[pl.ds((base_chunk + t) * CHUNK_BINS, CHUNK_BINS)], sem)

    issue_gather(0, rows0, gsem0)
    issue_gather(1, rows1, gsem1)

    @pl.loop(0, nchunks // 2)
    def _pair(u):
        for phase, rows, out_v, gsem, osem in (
                (0, rows0, out0, gsem0, osem0),
                (1, rows1, out1, gsem1, osem1)):
            t = u * 2 + phase
            wait_gather(rows, gsem)

            @pl.when(u > 0)
            def _():
                wait_out(out_v, osem)

            compute(t, rows, out_v)
            write_out(t, out_v, osem)

            @pl.when(t + 2 < nchunks)
            def _():
                issue_gather(t + 2, rows, gsem)

    @pl.when(nchunks % 2 == 1)
    def _tail():
        t = nchunks - 1
        wait_gather(rows0, gsem0)
        wait_out(out0, osem0)
        compute(t, rows0, out0)
        write_out(t, out0, osem0)

    wait_out(out0, osem0)
    wait_out(out1, osem1)


def _make_sc(total_bins, C):
    mesh = plsc.VectorSubcoreMesh(core_axis_name="c", subcore_axis_name="s")
    gpc = CHUNK_BINS * SLOTS
    total_chunks = total_bins // CHUNK_BINS
    max_chunks = (total_chunks + NUM_WORKERS - 1) // NUM_WORKERS
    spw = max_chunks * gpc
    cp = pltpu.CompilerParams()
    if "needs_layout_passes" in pltpu.CompilerParams.__dataclass_fields__:
        cp = dataclasses.replace(cp, needs_layout_passes=False)
    return pl.kernel(
        functools.partial(_sc_body, total_chunks, max_chunks),
        out_type=jax.ShapeDtypeStruct((total_bins, C), jnp.float32),
        mesh=mesh,
        scratch_types=[
            pltpu.VMEM((spw,), jnp.int32),
            pltpu.VMEM((spw + LANES,), jnp.float32),
            pltpu.VMEM((gpc, C // 2), jnp.int32),
            pltpu.VMEM((gpc, C // 2), jnp.int32),
            pltpu.VMEM((CHUNK_BINS, C), jnp.float32),
            pltpu.VMEM((CHUNK_BINS, C), jnp.float32),
            pltpu.SemaphoreType.DMA,
            pltpu.SemaphoreType.DMA,
            pltpu.SemaphoreType.DMA,
            pltpu.SemaphoreType.DMA,
        ],
        compiler_params=cp,
    )


def _pack_body(lo_ref, hi_ref, out_ref):
    def to_bits(v):
        b = lax.bitcast_convert_type(v.astype(jnp.bfloat16), jnp.int16)
        return b.astype(jnp.int32) & 0xFFFF

    out_ref[...] = to_bits(lo_ref[...]) | (to_bits(hi_ref[...]) << 16)


def _make_pack(NHW, C, BLK):
    hc = C // 2
    return pl.pallas_call(
        _pack_body,
        grid=(NHW // BLK,),
        in_specs=[
            pl.BlockSpec((BLK, hc), lambda i: (i, 0)),
            pl.BlockSpec((BLK, hc), lambda i: (i, 1)),
        ],
        out_specs=pl.BlockSpec((BLK, hc), lambda i: (i, 0)),
        out_shape=jax.ShapeDtypeStruct((NHW, hc), jnp.int32),
    )


def _pack_table(input, N, C, H, W):
    nhwc = jnp.transpose(input, (0, 2, 3, 1)).reshape(N * H * W, C)
    return _make_pack(N * H * W, C, 2000)(nhwc, nhwc)


def kernel(input, rois):
    N, C, H, W = input.shape
    R = rois.shape[0]
    nb = OUT_H * OUT_W
    bins = R * nb
    total_chunks = bins // CHUNK_BINS
    max_chunks = (total_chunks + NUM_WORKERS - 1) // NUM_WORKERS

    feat = _pack_table(input, N, C, H, W)
    cols = jnp.repeat(rois.T, SLOTS, axis=1)
    idx, wt = _make_idxw(R, N, H, W)(cols)
    pad = max_chunks * NUM_WORKERS * CHUNK_BINS * SLOTS - bins * SLOTS
    idx_bm = jnp.pad(idx.reshape(-1), (0, pad))
    wt_bm = jnp.pad(wt.reshape(-1), (0, pad))
    out_flat = _make_sc(bins, C)(feat, idx_bm, wt_bm)
    out = out_flat.reshape(OUT_H, OUT_W, R, C)
    return jnp.transpose(out, (2, 3, 0, 1))
